# baseline (speedup 1.0000x reference)
import jax
import jax.numpy as jnp
from jax.experimental import pallas as pl
from jax.experimental.pallas import tpu as pltpu

HID = 64
N_GRAPHS = 256
N_LAYERS = 4


def _head_body(gid_ref, hP_ref, hM_ref, w_ref, out_ref):
    gid = gid_ref[0, :]
    iota = jax.lax.broadcasted_iota(jnp.int32, (N_GRAPHS, gid.shape[0]), 0)
    onehot = (gid[None, :] == iota).astype(jnp.float32)
    gP = jnp.dot(onehot, hP_ref[...], preferred_element_type=jnp.float32)
    gM = jnp.dot(onehot, hM_ref[...], preferred_element_type=jnp.float32)

    W0 = w_ref[0:64, 0:32]
    b0 = w_ref[64:65, 0:32]
    W1 = w_ref[0:32, 32:48]
    b1 = w_ref[64:65, 32:48]
    W2 = w_ref[0:16, 48:56]
    b2 = w_ref[64:65, 48:56]
    W3 = w_ref[0:8, 56:60]
    b3 = w_ref[64:65, 56:60]
    oW = w_ref[0:4, 60:61]
    ob = w_ref[64:65, 60:61]

    def head(g):
        g = jax.nn.relu(jnp.dot(g, W0, preferred_element_type=jnp.float32) + b0)
        g = jax.nn.relu(jnp.dot(g, W1, preferred_element_type=jnp.float32) + b1)
        g = jax.nn.relu(jnp.dot(g, W2, preferred_element_type=jnp.float32) + b2)
        g = jax.nn.relu(jnp.dot(g, W3, preferred_element_type=jnp.float32) + b3)
        return jnp.dot(g, oW, preferred_element_type=jnp.float32) + ob

    oP = head(gP)
    oM = head(gM)
    out_ref[...] = oP - oM


def _run_head(graph_ids, hP, hM, lin_W0, lin_b0, lin_W1, lin_b1, lin_W2,
              lin_b2, lin_W3, lin_b3, out_W, out_b):
    N = hP.shape[0]
    Np = ((N + 127) // 128) * 128
    pad = Np - N
    gid = jnp.pad(graph_ids, (0, pad))[None, :]
    hPp = jnp.pad(hP, ((0, pad), (0, 0)))
    hMp = jnp.pad(hM, ((0, pad), (0, 0)))
    w = jnp.zeros((65, 61), jnp.float32)
    w = w.at[0:64, 0:32].set(lin_W0)
    w = w.at[64, 0:32].set(lin_b0)
    w = w.at[0:32, 32:48].set(lin_W1)
    w = w.at[64, 32:48].set(lin_b1)
    w = w.at[0:16, 48:56].set(lin_W2)
    w = w.at[64, 48:56].set(lin_b2)
    w = w.at[0:8, 56:60].set(lin_W3)
    w = w.at[64, 56:60].set(lin_b3)
    w = w.at[0:4, 60].set(out_W[:, 0])
    w = w.at[64, 60].set(out_b[0])
    return pl.pallas_call(
        _head_body,
        out_shape=jax.ShapeDtypeStruct((N_GRAPHS, 1), jnp.float32),
    )(gid, hPp, hMp, w)


def kernel(r_node, i_node, r_edge, d_edge, edge_index, ir_src, ir_dst,
           graph_ids, W_rnode, W_inode, W_edge, Wp, ap, Wm, am, Wd,
           lin_W0, lin_b0, lin_W1, lin_b1, lin_W2, lin_b2, lin_W3, lin_b3,
           out_W, out_b):
    N = r_node.shape[0]
    h0 = r_node @ W_rnode
    hi = i_node @ W_inode
    e = r_edge @ W_edge
    src = edge_index[0]
    dst = edge_index[1]
    d = d_edge @ Wd
    i_msg = jax.ops.segment_sum(hi[ir_src] * d, ir_dst, num_segments=N)

    def branch(h, Wst, ast, sign):
        for l in range(N_LAYERS):
            z = h[src] + h[dst] + e
            logits = sign * jax.nn.leaky_relu(z @ ast[l], negative_slope=0.01)
            mx = jax.ops.segment_max(logits, dst, num_segments=N)
            ex = jnp.exp(logits - mx[dst])
            den = jax.ops.segment_sum(ex, dst, num_segments=N)
            alpha = ex / (den[dst] + 1e-9)
            msg = (h[src] + e) * alpha[:, None]
            agg = jax.ops.segment_sum(msg, dst, num_segments=N) + i_msg
            h = jax.nn.leaky_relu(agg @ Wst[l], negative_slope=0.01) + h
        return h

    hP = branch(h0, Wp, ap, 1.0)
    hM = branch(h0, Wm, am, -1.0)

    return _run_head(graph_ids, hP, hM, lin_W0, lin_b0, lin_W1, lin_b1,
                     lin_W2, lin_b2, lin_W3, lin_b3, out_W, out_b)


# speedup vs baseline: 11.4013x; 11.4013x over previous
import jax
import jax.numpy as jnp
from jax import lax
from jax.experimental import pallas as pl
from jax.experimental.pallas import tpu as pltpu
from jax.experimental.pallas import tpu_sc as plsc

HID = 64
N_GRAPHS = 256
N_LAYERS = 4

_NC, _NS = 2, 16
_NW = _NC * _NS
_NSR = 256
_SRS = _NSR // _NS
_EIR_W = 5008


def _sc_mesh():
    return plsc.VectorSubcoreMesh(core_axis_name="c", subcore_axis_name="s")


def _row_iota():
    return jnp.arange(_NSR, dtype=jnp.int32).reshape(2, _NSR // 2)


def _zero_rows(ref, nrows):
    zeros16 = jnp.zeros((16,), jnp.float32)

    def body(r, carry):
        for k in range(4):
            ref[r, pl.ds(k * 16, 16)] = zeros16
        return carry

    lax.fori_loop(0, nrows, body, 0)


def _table_reduce_emit(acc_v, idx_v, shared, out_hbm, cid, sid):
    plsc.subcore_barrier()
    for j in range(2):
        pltpu.sync_copy(acc_v.at[pl.ds(j * (_NSR // 2), _NSR // 2), :],
                        shared.at[idx_v.at[j]], add=True)
    plsc.subcore_barrier()
    pltpu.sync_copy(shared.at[pl.ds(sid * _SRS, _SRS), :],
                    out_hbm.at[cid, pl.ds(sid * _SRS, _SRS), :])


def _zero_shared_stripe(acc_v, shared, sid):
    pltpu.sync_copy(acc_v.at[pl.ds(0, _SRS), :],
                    shared.at[pl.ds(sid * _SRS, _SRS), :])


def _sseg_body(inode_hbm, dedge_hbm, src_hbm, dst_hbm, ridx_hbm, out_hbm,
               inode_v, sv, dv, de_v, idx_v, acc_v, shared):
    cid = lax.axis_index("c")
    sid = lax.axis_index("s")
    wid = sid * _NC + cid
    base = wid * _EIR_W
    pltpu.sync_copy(inode_hbm, inode_v)
    pltpu.sync_copy(src_hbm.at[pl.ds(base, _EIR_W)], sv)
    pltpu.sync_copy(dst_hbm.at[pl.ds(base, _EIR_W)], dv)
    pltpu.sync_copy(dedge_hbm.at[pl.ds(base, _EIR_W)], de_v)
    pltpu.sync_copy(ridx_hbm, idx_v)
    _zero_rows(acc_v, _NSR)
    _zero_shared_stripe(acc_v, shared, sid)

    def ebody(g, carry):
        idx = sv[pl.ds(g * 16, 16)]
        didx = dv[pl.ds(g * 16, 16)]
        vals = plsc.load_gather(inode_v, [idx]) * de_v[pl.ds(g * 16, 16)]
        plsc.addupdate_scatter(acc_v, [didx >> 6, didx & 63], vals)
        return carry

    lax.fori_loop(0, _EIR_W // 16, ebody, 0)
    _table_reduce_emit(acc_v, idx_v, shared, out_hbm, cid, sid)


_EW = 10000
_K = 96
_EW2 = 10080
_NCH = _EW2 // _K
_NPN = 10240
_NST = _NPN // _NS


def _pass1_body(sign, u_hbm, ea_hbm, src_hbm, dst_hbm, ridx_hbm,
                ex_hbm, den_hbm, u_v, ea_v, sv, dv, ex_v, idx_v, acc_v,
                shared):
    cid = lax.axis_index("c")
    sid = lax.axis_index("s")
    wid = sid * _NC + cid
    base = wid * _EW
    pltpu.sync_copy(u_hbm, u_v)
    pltpu.sync_copy(ea_hbm.at[pl.ds(base, _EW)], ea_v)
    pltpu.sync_copy(src_hbm.at[pl.ds(base, _EW)], sv)
    pltpu.sync_copy(dst_hbm.at[pl.ds(base, _EW)], dv)
    pltpu.sync_copy(ridx_hbm, idx_v)
    _zero_rows(acc_v, _NSR)
    _zero_shared_stripe(acc_v, shared, sid)

    def ebody(g, carry):
        s16 = sv[pl.ds(g * 16, 16)]
        d16 = dv[pl.ds(g * 16, 16)]
        x = (plsc.load_gather(u_v, [s16]) + plsc.load_gather(u_v, [d16])
             + ea_v[pl.ds(g * 16, 16)])
        lr = jnp.maximum(x, 0.01 * x)
        ex = jnp.exp(lr) if sign > 0 else jnp.exp(-lr)
        ex_v[pl.ds(g * 16, 16)] = ex
        plsc.addupdate_scatter(acc_v, [d16 >> 6, d16 & 63], ex)
        return carry

    lax.fori_loop(0, _EW // 16, ebody, 0)
    pltpu.sync_copy(ex_v, ex_hbm.at[pl.ds(base, _EW)])
    _table_reduce_emit(acc_v, idx_v, shared, den_hbm, cid, sid)


def _make_pass1(sign, N, E):
    import functools
    return pl.kernel(
        functools.partial(_pass1_body, sign),
        out_type=(jax.ShapeDtypeStruct((E,), jnp.float32),
                  jax.ShapeDtypeStruct((_NC, _NSR, 64), jnp.float32)),
        mesh=_sc_mesh(),
        compiler_params=pltpu.CompilerParams(needs_layout_passes=False),
        scratch_types=[
            pltpu.VMEM((N,), jnp.float32),
            pltpu.VMEM((_EW,), jnp.float32),
            pltpu.VMEM((_EW,), jnp.int32),
            pltpu.VMEM((_EW,), jnp.int32),
            pltpu.VMEM((_EW,), jnp.float32),
            pltpu.VMEM((2, _NSR // 2), jnp.int32),
            pltpu.VMEM((_NSR, 64), jnp.float32),
            pltpu.VMEM_SHARED((_NSR, 64), jnp.float32),
        ],
    )


def _pass2_body(h_hbm, re_hbm, ex_hbm, src_hbm, dst2_hbm, den_hbm,
                t1_hbm, t16_hbm, sv, ex_v, dst2_v, d0_v, d1_v, rows_v,
                rr_v, zb1_v, zb16_v, avec, sem, t1_sh, t16_sh):
    cid = lax.axis_index("c")
    sid = lax.axis_index("s")
    wid = sid * _NC + cid
    base = wid * _EW2
    pltpu.sync_copy(src_hbm.at[pl.ds(base, _EW2)], sv)
    pltpu.sync_copy(ex_hbm.at[pl.ds(base, _EW2)], ex_v)
    pltpu.sync_copy(dst2_hbm.at[wid], dst2_v)
    pltpu.sync_copy(den_hbm.at[0], d0_v)
    pltpu.sync_copy(den_hbm.at[1], d1_v)

    def dbody(r, carry):
        for k in range(4):
            d0_v[r, pl.ds(k * 16, 16)] = (d0_v[r, pl.ds(k * 16, 16)]
                                          + d1_v[r, pl.ds(k * 16, 16)])
        return carry

    lax.fori_loop(0, _NSR, dbody, 0)

    _zero_rows(zb1_v, 80)
    zeros16 = jnp.zeros((16,), jnp.float32)

    def zb16body(r, carry):
        zb16_v[r, pl.ds(0, 16)] = zeros16
        return carry

    lax.fori_loop(0, 80, zb16body, 0)
    for j in range(_NST // 80):
        pltpu.sync_copy(zb1_v,
                        t1_sh.at[pl.ds(sid * _NST + j * 80, 80), :])
        pltpu.sync_copy(zb16_v,
                        t16_sh.at[pl.ds(sid * _NST + j * 80, 80), :])
    plsc.subcore_barrier()

    def cbody(c, carry):
        idxs = sv.at[pl.ds(c * _K, _K)]
        pltpu.async_copy(h_hbm.at[idxs], rows_v, sem).wait()
        pltpu.sync_copy(re_hbm.at[wid * _NCH + c], rr_v)

        def gbody(g, carry2):
            d16 = dst2_v[c, pl.ds(g * 16, 16)]
            ex16 = ex_v[pl.ds(c * _K + g * 16, 16)]
            den16 = plsc.load_gather(d0_v, [d16 >> 6, d16 & 63])
            avec[pl.ds(0, 16)] = ex16 / (den16 + 1e-9)
            for j in range(16):
                jidx = jnp.full((16,), j, dtype=jnp.int32)
                ab = plsc.load_gather(avec, [jidx])
                m = g * 16 + j
                for q in range(4):
                    rows_v[m, pl.ds(q * 16, 16)] = (
                        rows_v[m, pl.ds(q * 16, 16)] * ab)
                rr_v[m, pl.ds(0, 16)] = rr_v[m, pl.ds(0, 16)] * ab
            return carry2

        lax.fori_loop(0, _K // 16, gbody, 0)
        pltpu.sync_copy(rows_v, t1_sh.at[dst2_v.at[c]], add=True)
        pltpu.sync_copy(rr_v, t16_sh.at[dst2_v.at[c]], add=True)
        return carry

    lax.fori_loop(0, _NCH, cbody, 0)
    plsc.subcore_barrier()
    pltpu.sync_copy(t1_sh.at[pl.ds(sid * _NST, _NST), :],
                    t1_hbm.at[cid, pl.ds(sid * _NST, _NST), :])
    pltpu.sync_copy(t16_sh.at[pl.ds(sid * _NST, _NST), :],
                    t16_hbm.at[cid, pl.ds(sid * _NST, _NST), :])


def _make_pass2(N, Ep2):
    return pl.kernel(
        _pass2_body,
        out_type=(jax.ShapeDtypeStruct((_NC, _NPN, 64), jnp.float32),
                  jax.ShapeDtypeStruct((_NC, _NPN, 16), jnp.float32)),
        mesh=_sc_mesh(),
        compiler_params=pltpu.CompilerParams(
            needs_layout_passes=False, use_tc_tiling_on_sc=False),
        scratch_types=[
            pltpu.VMEM((_EW2,), jnp.int32),
            pltpu.VMEM((_EW2,), jnp.float32),
            pltpu.VMEM((_NCH, _K), jnp.int32),
            pltpu.VMEM((_NSR, 64), jnp.float32),
            pltpu.VMEM((_NSR, 64), jnp.float32),
            pltpu.VMEM((_K, 64), jnp.float32),
            pltpu.VMEM((_K, 16), jnp.float32),
            pltpu.VMEM((80, 64), jnp.float32),
            pltpu.VMEM((80, 16), jnp.float32),
            pltpu.VMEM((16,), jnp.float32),
            pltpu.SemaphoreType.DMA,
            pltpu.VMEM_SHARED((_NPN, 64), jnp.float32),
            pltpu.VMEM_SHARED((_NPN, 16), jnp.float32),
        ],
    )


def _run_sseg(i_node, d_edge, ir_src, ir_dst):
    N = i_node.shape[0]
    E = ir_src.shape[0]
    Ep = _NW * _EIR_W
    srcp = jnp.pad(ir_src, (0, Ep - E))
    dstp = jnp.pad(ir_dst, (0, Ep - E))
    dep = jnp.pad(d_edge[:, 0], (0, Ep - E))
    f = pl.kernel(
        _sseg_body,
        out_type=jax.ShapeDtypeStruct((_NC, _NSR, 64), jnp.float32),
        mesh=_sc_mesh(),
        compiler_params=pltpu.CompilerParams(needs_layout_passes=False),
        scratch_types=[
            pltpu.VMEM((N,), jnp.float32),
            pltpu.VMEM((_EIR_W,), jnp.int32),
            pltpu.VMEM((_EIR_W,), jnp.int32),
            pltpu.VMEM((_EIR_W,), jnp.float32),
            pltpu.VMEM((2, _NSR // 2), jnp.int32),
            pltpu.VMEM((_NSR, 64), jnp.float32),
            pltpu.VMEM_SHARED((_NSR, 64), jnp.float32),
        ],
    )
    out = f(i_node[:, 0], dep, srcp, dstp, _row_iota())
    out = out.reshape(_NC, _NSR * 64)
    return out[0, :N] + out[1, :N]


def _head_body(gid_ref, hP_ref, hM_ref, w_ref, out_ref):
    gid = gid_ref[0, :]
    iota = jax.lax.broadcasted_iota(jnp.int32, (N_GRAPHS, gid.shape[0]), 0)
    onehot = (gid[None, :] == iota).astype(jnp.float32)
    gP = jnp.dot(onehot, hP_ref[...], preferred_element_type=jnp.float32)
    gM = jnp.dot(onehot, hM_ref[...], preferred_element_type=jnp.float32)

    W0 = w_ref[0:64, 0:32]
    b0 = w_ref[64:65, 0:32]
    W1 = w_ref[0:32, 32:48]
    b1 = w_ref[64:65, 32:48]
    W2 = w_ref[0:16, 48:56]
    b2 = w_ref[64:65, 48:56]
    W3 = w_ref[0:8, 56:60]
    b3 = w_ref[64:65, 56:60]
    oW = w_ref[0:4, 60:61]
    ob = w_ref[64:65, 60:61]

    def head(g):
        g = jax.nn.relu(jnp.dot(g, W0, preferred_element_type=jnp.float32) + b0)
        g = jax.nn.relu(jnp.dot(g, W1, preferred_element_type=jnp.float32) + b1)
        g = jax.nn.relu(jnp.dot(g, W2, preferred_element_type=jnp.float32) + b2)
        g = jax.nn.relu(jnp.dot(g, W3, preferred_element_type=jnp.float32) + b3)
        return jnp.dot(g, oW, preferred_element_type=jnp.float32) + ob

    oP = head(gP)
    oM = head(gM)
    out_ref[...] = oP - oM


def _run_head(graph_ids, hP, hM, lin_W0, lin_b0, lin_W1, lin_b1, lin_W2,
              lin_b2, lin_W3, lin_b3, out_W, out_b):
    N = hP.shape[0]
    Np = ((N + 127) // 128) * 128
    pad = Np - N
    gid = jnp.pad(graph_ids, (0, pad))[None, :]
    hPp = jnp.pad(hP, ((0, pad), (0, 0)))
    hMp = jnp.pad(hM, ((0, pad), (0, 0)))
    w = jnp.zeros((65, 61), jnp.float32)
    w = w.at[0:64, 0:32].set(lin_W0)
    w = w.at[64, 0:32].set(lin_b0)
    w = w.at[0:32, 32:48].set(lin_W1)
    w = w.at[64, 32:48].set(lin_b1)
    w = w.at[0:16, 48:56].set(lin_W2)
    w = w.at[64, 48:56].set(lin_b2)
    w = w.at[0:8, 56:60].set(lin_W3)
    w = w.at[64, 56:60].set(lin_b3)
    w = w.at[0:4, 60].set(out_W[:, 0])
    w = w.at[64, 60].set(out_b[0])
    return pl.pallas_call(
        _head_body,
        out_shape=jax.ShapeDtypeStruct((N_GRAPHS, 1), jnp.float32),
    )(gid, hPp, hMp, w)


def kernel(r_node, i_node, r_edge, d_edge, edge_index, ir_src, ir_dst,
           graph_ids, W_rnode, W_inode, W_edge, Wp, ap, Wm, am, Wd,
           lin_W0, lin_b0, lin_W1, lin_b1, lin_W2, lin_b2, lin_W3, lin_b3,
           out_W, out_b):
    N = r_node.shape[0]
    E = edge_index.shape[1]
    h0 = r_node @ W_rnode
    src = edge_index[0]
    dst = edge_index[1]
    sseg = _run_sseg(i_node, d_edge, ir_src, ir_dst)
    i_msg = sseg[:, None] * (W_inode[0] * Wd[0])[None, :]

    a_all = jnp.concatenate([ap, am], axis=0)
    eaT = (r_edge @ (W_edge @ a_all.T)).T

    Ep2 = _NW * _EW2
    srcp2 = jnp.pad(src, (0, Ep2 - E))
    dst2 = jnp.pad(dst, (0, Ep2 - E)).reshape(_NW, _NCH, _K)
    rep2 = jnp.pad(r_edge, ((0, Ep2 - E), (0, 0))).reshape(_NW * _NCH, _K, 16)
    ridx = _row_iota()

    p1p = _make_pass1(1.0, N, E)
    p1m = _make_pass1(-1.0, N, E)
    p2 = _make_pass2(N, Ep2)

    def branch(h, Wst, ast, p1, a_off):
        for l in range(N_LAYERS):
            u = h @ ast[l]
            ex, den = p1(u, eaT[a_off + l], src, dst, ridx)
            exp2 = jnp.pad(ex, (0, Ep2 - E))
            t1p, t16p = p2(h, rep2, exp2, srcp2, dst2, den)
            t1 = t1p[0, :N] + t1p[1, :N]
            t16 = t16p[0, :N] + t16p[1, :N]
            agg = t1 + t16 @ W_edge + i_msg
            h = jax.nn.leaky_relu(agg @ Wst[l], negative_slope=0.01) + h
        return h

    hP = branch(h0, Wp, ap, p1p, 0)
    hM = branch(h0, Wm, am, p1m, 4)

    return _run_head(graph_ids, hP, hM, lin_W0, lin_b0, lin_W1, lin_b1,
                     lin_W2, lin_b2, lin_W3, lin_b3, out_W, out_b)
